# baseline (speedup 1.0000x reference)
import functools

import jax
import jax.numpy as jnp
from jax import lax
from jax.experimental import pallas as pl
from jax.experimental.pallas import tpu as pltpu
from jax.experimental.pallas import tpu_sc as plsc

LATENT_DIM = 128

_mesh = plsc.VectorSubcoreMesh(core_axis_name="c", subcore_axis_name="s")


@functools.partial(
    pl.kernel,
    mesh=_mesh,
    out_type=jax.ShapeDtypeStruct((1, LATENT_DIM), jnp.float32),
    scratch_types=[
        pltpu.VMEM((1, LATENT_DIM), jnp.float32),
    ],
)
def _sc_row_copy(w_hbm, out_hbm, row_v):
    wid = lax.axis_index("s") * 2 + lax.axis_index("c")

    @pl.when(wid == 0)
    def _():
        pltpu.sync_copy(w_hbm, row_v)
        pltpu.sync_copy(row_v, out_hbm)


def kernel(user_emb_weight):
    return _sc_row_copy(user_emb_weight)


# speedup vs baseline: 1.1787x; 1.1787x over previous
import functools

import jax
import jax.numpy as jnp
from jax import lax
from jax.experimental import pallas as pl
from jax.experimental.pallas import tpu as pltpu
from jax.experimental.pallas import tpu_sc as plsc

LATENT_DIM = 128

_mesh = plsc.ScalarSubcoreMesh(axis_name="c", num_cores=1)


@functools.partial(
    pl.kernel,
    mesh=_mesh,
    out_type=jax.ShapeDtypeStruct((1, LATENT_DIM), jnp.float32),
)
def _sc_row_copy(w_hbm, out_hbm):
    pltpu.sync_copy(w_hbm, out_hbm)


def kernel(user_emb_weight):
    return _sc_row_copy(user_emb_weight)
